# baseline (speedup 1.0000x reference)
import functools

import jax
import jax.numpy as jnp
from jax import lax
from jax.experimental import pallas as pl
from jax.experimental.pallas import tpu as pltpu

SEQ = 32768
VOCAB = 5
VPAD = 8
D = 384
RB = 512
NBLK = SEQ // RB


def _tc_body(ids_ref, table_ref, out_ref):
    idb = ids_ref[0, 0, :]
    onehot = (idb[:, None] == lax.broadcasted_iota(jnp.int32, (RB, VPAD), 1))
    out_ref[...] = jnp.dot(onehot.astype(jnp.float32), table_ref[...],
                           preferred_element_type=jnp.float32)


_tc_expand = pl.pallas_call(
    _tc_body,
    grid=(NBLK,),
    in_specs=[
        pl.BlockSpec((1, 1, RB), lambda i: (i, 0, 0)),
        pl.BlockSpec((VPAD, D), lambda i: (0, 0)),
    ],
    out_specs=pl.BlockSpec((RB, D), lambda i: (i, 0)),
    out_shape=jax.ShapeDtypeStruct((SEQ, D), jnp.float32),
    compiler_params=pltpu.CompilerParams(
        dimension_semantics=("parallel",)),
)


def kernel(ids, table):
    ids3 = ids.astype(jnp.int32).reshape(NBLK, 1, RB)
    tpad = jnp.pad(table, ((0, VPAD - VOCAB), (0, 0)))
    return _tc_expand(ids3, tpad)
